# baseline (speedup 1.0000x reference)
import functools

import jax
import jax.numpy as jnp
from jax.experimental import pallas as pl
from jax.experimental.pallas import tpu as pltpu


def _se_fused_kernel(x_ref, wt_ref, o_ref, *, inv_hw):
    x = x_ref[...]
    mean = jnp.sum(x, axis=1, dtype=jnp.float32) * inv_hw
    z = jnp.dot(mean, wt_ref[...], preferred_element_type=jnp.float32)
    gate = jax.nn.sigmoid(z).astype(x.dtype)
    o_ref[...] = x * gate[:, None, :]


def kernel(x_nchw, weight):
    N, C, H, W = x_nchw.shape
    HW = H * W
    x = jnp.transpose(x_nchw, (0, 2, 3, 1)).reshape(N, HW, C)
    w_t = weight.astype(jnp.float32).T

    body = functools.partial(_se_fused_kernel, inv_hw=float(1.0 / HW))

    itemsize = jnp.dtype(x.dtype).itemsize
    cost = pl.CostEstimate(
        flops=3 * N * C * HW + 2 * N * C * C,
        transcendentals=N * C,
        bytes_accessed=2 * N * C * HW * itemsize + C * C * 4,
    )
    B = 4 if N % 4 == 0 else (2 if N % 2 == 0 else 1)
    out = pl.pallas_call(
        body,
        out_shape=jax.ShapeDtypeStruct((N, HW, C), x.dtype),
        grid=(N // B,),
        in_specs=[
            pl.BlockSpec((B, HW, C), lambda n: (n, 0, 0)),
            pl.BlockSpec((C, C), lambda n: (0, 0)),
        ],
        out_specs=pl.BlockSpec((B, HW, C), lambda n: (n, 0, 0)),
        compiler_params=pltpu.CompilerParams(
            dimension_semantics=("parallel",),
            vmem_limit_bytes=64 * 1024 * 1024,
        ),
        cost_estimate=cost,
    )(x, w_t)
    return jnp.transpose(out.reshape(N, H, W, C), (0, 3, 1, 2))
